# baseline (speedup 1.0000x reference)
import functools

import jax
import jax.numpy as jnp
from jax import lax
from jax.experimental import pallas as pl
from jax.experimental.pallas import tpu as pltpu
from jax.experimental.pallas import tpu_sc as plsc

X_MAX = 5.0
ALPHA = 0.75

NC = 2
NS = 16
L = 16
NW = NC * NS


def _partials_call(i, j, l_vecs, r_vecs, lb1, rb1):
    B = i.shape[0]
    E = l_vecs.shape[1]
    BPW = B // NW
    GROUPS = BPW // L

    mesh = plsc.VectorSubcoreMesh(core_axis_name="c", subcore_axis_name="s")

    @functools.partial(
        pl.kernel,
        out_type=jax.ShapeDtypeStruct((NW, 4, L), jnp.float32),
        mesh=mesh,
        compiler_params=pltpu.CompilerParams(
            needs_layout_passes=False, use_tc_tiling_on_sc=False),
        scratch_types=[
            pltpu.VMEM((BPW,), jnp.int32),
            pltpu.VMEM((BPW,), jnp.int32),
            pltpu.VMEM((BPW, E), jnp.float32),
            pltpu.VMEM((BPW, E), jnp.float32),
            pltpu.VMEM((BPW, 1), jnp.float32),
            pltpu.VMEM((BPW, 1), jnp.float32),
            pltpu.VMEM((4, L), jnp.float32),
            pltpu.SemaphoreType.DMA,
            pltpu.SemaphoreType.DMA,
            pltpu.SemaphoreType.DMA,
        ],
    )
    def k(i_hbm, j_hbm, lv_hbm, rv_hbm, lb_hbm, rb_hbm, out_hbm,
          idx_i, idx_j, l_rows, r_rows, lb_v, rb_v, partials,
          sem0, sem1, sem2):
        wid = lax.axis_index("s") * NC + lax.axis_index("c")
        base = wid * BPW
        pltpu.sync_copy(i_hbm.at[pl.ds(base, BPW)], idx_i)
        pltpu.sync_copy(j_hbm.at[pl.ds(base, BPW)], idx_j)

        sems = (sem0, sem1)

        def issue_group(g):
            sem = sems[g % 2]
            ich = idx_i[pl.ds(g * L, L)]
            jch = idx_j[pl.ds(g * L, L)]
            for u in range(L):
                p = g * L + u
                v = ich[u]
                w = jch[u]
                pltpu.async_copy(lv_hbm.at[pl.ds(v, 1), :],
                                 l_rows.at[pl.ds(p, 1), :], sem)
                pltpu.async_copy(rv_hbm.at[pl.ds(w, 1), :],
                                 r_rows.at[pl.ds(p, 1), :], sem)
                pltpu.async_copy(lb_hbm.at[pl.ds(v, 1), :],
                                 lb_v.at[pl.ds(p, 1), :], sem)
                pltpu.async_copy(rb_hbm.at[pl.ds(w, 1), :],
                                 rb_v.at[pl.ds(p, 1), :], sem)

        def drain_group(g):
            sem = sems[g % 2]
            s = pl.ds(g * L, L)
            pltpu.make_async_copy(lv_hbm.at[pl.ds(0, L), :],
                                  l_rows.at[s, :], sem).wait()
            pltpu.make_async_copy(rv_hbm.at[pl.ds(0, L), :],
                                  r_rows.at[s, :], sem).wait()
            pltpu.make_async_copy(lb_hbm.at[pl.ds(0, L), :],
                                  lb_v.at[s, :], sem).wait()
            pltpu.make_async_copy(rb_hbm.at[pl.ds(0, L), :],
                                  rb_v.at[s, :], sem).wait()

        iota = lax.iota(jnp.int32, L)
        zf = jnp.zeros((L,), jnp.float32)
        Sd, Sd2, Se, Se2 = zf, zf, zf, zf

        issue_group(0)
        for g in range(GROUPS):
            if g + 1 < GROUPS:
                issue_group(g + 1)
            drain_group(g)
            pvec = iota + (g * L)
            s = pl.ds(g * L, L)

            def kbody(kk, acc, pvec=pvec):
                for u in range(4):
                    kv = jnp.full((L,), kk * 4 + u, jnp.int32)
                    a = plsc.load_gather(l_rows, [pvec, kv])
                    b = plsc.load_gather(r_rows, [pvec, kv])
                    acc = acc + a * b
                return acc

            d = lax.fori_loop(0, E // 4, kbody, zf)
            zero16 = iota * 0
            e = (plsc.load_gather(lb_v, [pvec, zero16]) +
                 plsc.load_gather(rb_v, [pvec, zero16]))
            Sd = Sd + d
            Sd2 = Sd2 + d * d
            Se = Se + e
            Se2 = Se2 + e * e
        partials[0, :] = Sd
        partials[1, :] = Sd2
        partials[2, :] = Se
        partials[3, :] = Se2
        pltpu.sync_copy(partials, out_hbm.at[wid])

    return k(i, j, l_vecs, r_vecs, lb1, rb1)


def kernel(i, j, count, l_vecs, r_vecs, l_bias, r_bias):
    B = i.shape[0]
    parts = _partials_call(i, j, l_vecs, r_vecs, l_bias, r_bias)
    s = parts.sum(axis=(0, 2))
    Sd, Sd2, Se, Se2 = s[0], s[1], s[2], s[3]
    c0 = count[0]
    logc = jnp.log(c0)
    wfn = jnp.where(c0 < X_MAX, (c0 / X_MAX) ** ALPHA, jnp.float32(1.0))
    Bf = jnp.float32(B)
    Sc = Se - Bf * logc
    Sc2 = Se2 - 2.0 * logc * Se + Bf * logc * logc
    return wfn * (Bf * Sd2 + 2.0 * Sd * Sc + Bf * Sc2)


# speedup vs baseline: 2.2200x; 2.2200x over previous
import functools

import jax
import jax.numpy as jnp
from jax import lax
from jax.experimental import pallas as pl
from jax.experimental.pallas import tpu as pltpu
from jax.experimental.pallas import tpu_sc as plsc

X_MAX = 5.0
ALPHA = 0.75

NC = 2
NS = 16
L = 16
NW = NC * NS


def _partials_call(i, j, l_vecs, r_vecs, lb_flat, rb_flat):
    B = i.shape[0]
    E = l_vecs.shape[1]
    BPW = B // NW
    GROUPS = BPW // L

    mesh = plsc.VectorSubcoreMesh(core_axis_name="c", subcore_axis_name="s")

    @functools.partial(
        pl.kernel,
        out_type=jax.ShapeDtypeStruct((NW, 4, L), jnp.float32),
        mesh=mesh,
        compiler_params=pltpu.CompilerParams(
            needs_layout_passes=False, use_tc_tiling_on_sc=False),
        scratch_types=[
            pltpu.VMEM((BPW,), jnp.int32),
            pltpu.VMEM((BPW,), jnp.int32),
            pltpu.VMEM((BPW, E), jnp.float32),
            pltpu.VMEM((BPW, E), jnp.float32),
            pltpu.VMEM((BPW,), jnp.float32),
            pltpu.VMEM((BPW,), jnp.float32),
            pltpu.VMEM((4, L), jnp.float32),
            pltpu.SemaphoreType.DMA,
            pltpu.SemaphoreType.DMA,
            pltpu.SemaphoreType.DMA,
            pltpu.SemaphoreType.DMA,
        ],
    )
    def k(i_hbm, j_hbm, lv_hbm, rv_hbm, lb_hbm, rb_hbm, out_hbm,
          idx_i, idx_j, l_rows, r_rows, lb_v, rb_v, partials,
          sem0, sem1, sem2, sem3):
        wid = lax.axis_index("s") * NC + lax.axis_index("c")
        base = wid * BPW
        pltpu.sync_copy(i_hbm.at[pl.ds(base, BPW)], idx_i)
        pltpu.sync_copy(j_hbm.at[pl.ds(base, BPW)], idx_j)
        cp0 = pltpu.async_copy(lv_hbm.at[idx_i], l_rows, sem0)
        cp1 = pltpu.async_copy(rv_hbm.at[idx_j], r_rows, sem1)
        cp2 = pltpu.async_copy(lb_hbm.at[idx_i], lb_v, sem2)
        cp3 = pltpu.async_copy(rb_hbm.at[idx_j], rb_v, sem3)
        cp0.wait()
        cp1.wait()
        cp2.wait()
        cp3.wait()

        iota = lax.iota(jnp.int32, L)
        zf = jnp.zeros((L,), jnp.float32)
        Sd, Sd2, Se, Se2 = zf, zf, zf, zf
        for g in range(GROUPS):
            pvec = iota + (g * L)

            def kbody(kk, acc, pvec=pvec):
                for u in range(4):
                    kv = jnp.full((L,), kk * 4 + u, jnp.int32)
                    a = plsc.load_gather(l_rows, [pvec, kv])
                    b = plsc.load_gather(r_rows, [pvec, kv])
                    acc = acc + a * b
                return acc

            d = lax.fori_loop(0, E // 4, kbody, zf)
            e = lb_v[pl.ds(g * L, L)] + rb_v[pl.ds(g * L, L)]
            Sd = Sd + d
            Sd2 = Sd2 + d * d
            Se = Se + e
            Se2 = Se2 + e * e
        partials[0, :] = Sd
        partials[1, :] = Sd2
        partials[2, :] = Se
        partials[3, :] = Se2
        pltpu.sync_copy(partials, out_hbm.at[wid])

    return k(i, j, l_vecs, r_vecs, lb_flat, rb_flat)


def kernel(i, j, count, l_vecs, r_vecs, l_bias, r_bias):
    B = i.shape[0]
    parts = _partials_call(i, j, l_vecs, r_vecs,
                           l_bias.reshape(-1), r_bias.reshape(-1))
    s = parts.sum(axis=(0, 2))
    Sd, Sd2, Se, Se2 = s[0], s[1], s[2], s[3]
    c0 = count[0]
    logc = jnp.log(c0)
    wfn = jnp.where(c0 < X_MAX, (c0 / X_MAX) ** ALPHA, jnp.float32(1.0))
    Bf = jnp.float32(B)
    Sc = Se - Bf * logc
    Sc2 = Se2 - 2.0 * logc * Se + Bf * logc * logc
    return wfn * (Bf * Sd2 + 2.0 * Sd * Sc + Bf * Sc2)
